# baseline (speedup 1.0000x reference)
import functools

import jax
import jax.numpy as jnp
from jax import lax
from jax.experimental import pallas as pl
import jax.experimental.pallas.tpu as pltpu
from jax.experimental.pallas import tpu_sc as plsc

_B, _C, _N = 128, 32768, 8
_BB = 8
_NW = 32
_JPW = _B * _N // _NW
_ZCOLS = 2048
_NZ = _C // _ZCOLS

_sc_mesh = plsc.VectorSubcoreMesh(core_axis_name="c", subcore_axis_name="s")


@functools.partial(
    pl.kernel,
    mesh=_sc_mesh,
    out_type=jax.ShapeDtypeStruct((_B * _N, _C), jnp.float32),
    scratch_types=[
        pltpu.VMEM((_JPW, _ZCOLS), jnp.float32),
        pltpu.SemaphoreType.DMA,
    ],
)
def _sc_zero(out_hbm, zbuf, sem):
    w = lax.axis_index("s") * 2 + lax.axis_index("c")
    zero16 = jnp.zeros((16,), jnp.float32)

    def zrow(i, carry):
        for s in range(_ZCOLS // 16):
            zbuf[i, pl.ds(s * 16, 16)] = zero16
        return carry

    lax.fori_loop(0, _JPW, zrow, 0)
    base = w * _JPW
    copies = [
        pltpu.make_async_copy(
            zbuf,
            out_hbm.at[pl.ds(base, _JPW), pl.ds(k * _ZCOLS, _ZCOLS)],
            sem,
        )
        for k in range(_NZ)
    ]
    for cpy in copies:
        cpy.start()
    for cpy in copies:
        cpy.wait()


def _argmax_body(x_ref, p_ref):
    xb = x_ref[...]
    colmax = jnp.max(xb, axis=2, keepdims=True)
    colmin = jnp.min(xb, axis=2, keepdims=True)
    cidx = lax.broadcasted_iota(jnp.int32, xb.shape, 2)
    idxm = jnp.where(xb == colmax, cidx, jnp.int32(1 << 30))
    amin = jnp.min(idxm, axis=2, keepdims=True)
    valid = (colmax != colmin).astype(jnp.int32) << 16
    packed = amin | valid
    p_ref[...] = jnp.broadcast_to(packed, (_BB, _N, 128))


@functools.partial(
    pl.kernel,
    mesh=_sc_mesh,
    out_type=(),
    scratch_types=[
        pltpu.VMEM((_JPW // 8, _N, 128), jnp.int32),
        pltpu.VMEM((_JPW, 128), jnp.float32),
        pltpu.SemaphoreType.DMA,
    ],
)
def _sc_scatter(packed_hbm, out_hbm, cst_v, src_v, sem):
    w = lax.axis_index("s") * 2 + lax.axis_index("c")
    bpw = _JPW // 8
    pltpu.sync_copy(packed_hbm.at[pl.ds(w * bpw, bpw)], cst_v)
    iota = lax.iota(jnp.int32, 16)
    copies = []
    for b_loc in range(bpw):
        for n in range(_N):
            row = b_loc * _N + n
            cvec = cst_v[b_loc, n, pl.ds(0, 16)]
            c = cvec[0]
            cstar = c & 0xFFFF
            lane = c & 127
            valf = jnp.where(
                (c >> 16) == 1, jnp.float32(1.0), jnp.float32(0.0)
            )
            for s in range(8):
                vals = jnp.where(iota + 16 * s == lane, valf, 0.0)
                src_v[row, pl.ds(s * 16, 16)] = vals
            j = (w * bpw + b_loc) * _N + n
            c0 = (cstar >> 7) * 128
            copies.append(
                pltpu.make_async_copy(
                    src_v.at[pl.ds(row, 1)],
                    out_hbm.at[pl.ds(j, 1), pl.ds(c0, 128)],
                    sem,
                )
            )
    for cpy in copies:
        cpy.start()
    for cpy in copies:
        cpy.wait()


def kernel(x):
    B, C, N = x.shape
    xt = jnp.transpose(x, (0, 2, 1))
    packed = pl.pallas_call(
        _argmax_body,
        grid=(B // _BB,),
        in_specs=[pl.BlockSpec((_BB, N, C), lambda b: (b, 0, 0))],
        out_specs=pl.BlockSpec((_BB, N, 128), lambda b: (b, 0, 0)),
        out_shape=jax.ShapeDtypeStruct((B, N, 128), jnp.int32),
    )(xt)
    zbuf = _sc_zero()
    out_ref = jax.new_ref(zbuf)
    _sc_scatter(packed, out_ref)
    out2d = out_ref[...]
    return jnp.transpose(out2d.reshape(B, N, C), (0, 2, 1))


# speedup vs baseline: 1.2162x; 1.2162x over previous
import jax
import jax.numpy as jnp
from jax.experimental import pallas as pl


_BB = 8


def _onehot_body(x_ref, o_ref):
    xb = x_ref[...]
    colmax = jnp.max(xb, axis=2, keepdims=True)
    colmin = jnp.min(xb, axis=2, keepdims=True)

    cidx = jax.lax.broadcasted_iota(jnp.int32, xb.shape, 2)
    big = jnp.int32(1 << 30)
    idxm = jnp.where(xb == colmax, cidx, big)
    amin = jnp.min(idxm, axis=2, keepdims=True)

    valid = colmax != colmin
    one = (cidx == amin) & valid
    o_ref[...] = one.astype(jnp.float32)


def kernel(x):
    B, C, N = x.shape
    xt = jnp.transpose(x, (0, 2, 1))
    out = pl.pallas_call(
        _onehot_body,
        grid=(B // _BB,),
        in_specs=[pl.BlockSpec((_BB, N, C), lambda b: (b, 0, 0))],
        out_specs=pl.BlockSpec((_BB, N, C), lambda b: (b, 0, 0)),
        out_shape=jax.ShapeDtypeStruct((B, N, C), jnp.float32),
    )(xt)
    return jnp.transpose(out, (0, 2, 1))


# speedup vs baseline: 1.2200x; 1.0031x over previous
import jax
import jax.numpy as jnp
from jax.experimental import pallas as pl
import jax.experimental.pallas.tpu as pltpu


_BB = 8


def _onehot_body(x_ref, o_ref):
    xb = x_ref[...]
    colmax = jnp.max(xb, axis=2, keepdims=True)
    colmin = jnp.min(xb, axis=2, keepdims=True)

    cidx = jax.lax.broadcasted_iota(jnp.int32, xb.shape, 2)
    big = jnp.int32(1 << 30)
    idxm = jnp.where(xb == colmax, cidx, big)
    amin = jnp.min(idxm, axis=2, keepdims=True)

    valid = colmax != colmin
    one = (cidx == amin) & valid
    o_ref[...] = one.astype(jnp.float32)


def kernel(x):
    B, C, N = x.shape
    xt = jnp.transpose(x, (0, 2, 1))
    out = pl.pallas_call(
        _onehot_body,
        grid=(B // _BB,),
        in_specs=[pl.BlockSpec((_BB, N, C), lambda b: (b, 0, 0))],
        out_specs=pl.BlockSpec((_BB, N, C), lambda b: (b, 0, 0)),
        out_shape=jax.ShapeDtypeStruct((B, N, C), jnp.float32),
        compiler_params=pltpu.CompilerParams(
            dimension_semantics=("parallel",),
        ),
    )(xt)
    return jnp.transpose(out, (0, 2, 1))
